# baseline (speedup 1.0000x reference)
import functools

import jax
import jax.numpy as jnp
from jax.experimental import pallas as pl
from jax.experimental.pallas import tpu as pltpu


def _batch_loss(raw, g, F, Pp, T, N):
    s = jnp.clip(g[:, 0:1], 0.0, 1.0)
    e = jnp.clip(g[:, 1:2], 0.0, 1.0)

    lane = jax.lax.broadcasted_iota(jnp.int32, (1, F), 1).astype(jnp.float32)
    left = lane * (1.0 / F)
    right = left + (1.0 / F)
    ov = jnp.clip(jnp.minimum(e, right) - jnp.maximum(s, left), 0.0, None)
    not_skip = jnp.sum(ov) > 0.0
    n_col = jax.lax.broadcasted_iota(jnp.int32, (N, F), 0)
    s_b = jnp.broadcast_to(s, (N, F))
    e_b = jnp.broadcast_to(e, (N, F))

    main_b = jnp.float32(0.0)
    npos_b = jnp.float32(0.0)
    for p in range(Pp):
        m = jnp.max(ov, axis=0, keepdims=True)
        idx = jnp.min(jnp.where(ov == m, n_col, N), axis=0, keepdims=True)
        oh = n_col == idx
        ts = jnp.sum(jnp.where(oh, s_b, 0.0), axis=0, keepdims=True)
        te = jnp.sum(jnp.where(oh, e_b, 0.0), axis=0, keepdims=True)
        pos = (m > 0.0) & not_skip
        ov = jnp.where(oh, -1.0, ov)

        z = pos.astype(jnp.float32)
        rw = 5.0 * z
        aw = 0.5 + 0.5 * z

        ps = raw[p, 0]
        pe = raw[p, 1]
        pc = raw[p, 2]
        d1 = jnp.abs(ps - ts)
        m1 = jnp.minimum(d1, 1.0)
        d2 = jnp.abs(pe - te)
        m2 = jnp.minimum(d2, 1.0)
        sl = m1 * (d1 - 0.5 * m1) + m2 * (d2 - 0.5 * m2)
        sp = jnp.maximum(pc, 0.0) + jnp.log1p(jnp.exp(-jnp.abs(pc)))
        contrib = rw * sl + aw * sp - z * pc
        main_b += jnp.sum(contrib)
        npos_b += jnp.float32(T) * jnp.sum(z)
    return main_b, npos_b


def _loss_kernel(gt_ref, raw_hbm, out_ref, *scratch):
    B = raw_hbm.shape[0]
    _, Pp, C, T, F = raw_hbm.shape
    N = gt_ref.shape[1]
    bufs = scratch[:B]
    sems = scratch[B:]

    copies = [
        pltpu.make_async_copy(raw_hbm.at[b], bufs[b], sems[b])
        for b in range(B)
    ]
    LOOKAHEAD = 8
    for b in range(LOOKAHEAD):
        copies[b].start()

    main = jnp.float32(0.0)
    npos = jnp.float32(0.0)
    for b in range(B):
        copies[b].wait()
        if b + LOOKAHEAD < B:
            copies[b + LOOKAHEAD].start()
        m_b, n_b = _batch_loss(bufs[b][...], gt_ref[b], F, Pp, T, N)
        main += m_b
        npos += n_b

    out_ref[...] = jnp.concatenate(
        [jnp.full((1, 128), main, jnp.float32),
         jnp.full((1, 128), npos, jnp.float32),
         jnp.zeros((6, 128), jnp.float32)], axis=0)


@functools.partial(jax.jit, static_argnames=())
def kernel(raw_preds, gt_boxes):
    B, Pp, C, T, F = raw_preds.shape
    N = gt_boxes.shape[1]
    out = pl.pallas_call(
        _loss_kernel,
        in_specs=[
            pl.BlockSpec(memory_space=pltpu.MemorySpace.VMEM),
            pl.BlockSpec(memory_space=pl.ANY),
        ],
        out_specs=pl.BlockSpec(memory_space=pltpu.MemorySpace.VMEM),
        out_shape=jax.ShapeDtypeStruct((8, 128), jnp.float32),
        scratch_shapes=(
            [pltpu.VMEM((Pp, C, T, F), jnp.float32) for _ in range(B)]
            + [pltpu.SemaphoreType.DMA for _ in range(B)]),
    )(gt_boxes, raw_preds)
    main = out[0, 0]
    n_pos = out[1, 0]
    return main / jnp.maximum(n_pos, 1.0)


# speedup vs baseline: 1.1211x; 1.1211x over previous
import functools

import jax
import jax.numpy as jnp
from jax.experimental import pallas as pl


def _loss_block(raw_ref, gt_ref, out_ref):
    b = pl.program_id(0)
    BB, Pp, _, T, F = raw_ref.shape
    N = gt_ref.shape[1]

    main_b = jnp.float32(0.0)
    npos_b = jnp.float32(0.0)
    for bb in range(BB):
        g = gt_ref[bb]
        s = jnp.clip(g[:, 0:1], 0.0, 1.0)
        e = jnp.clip(g[:, 1:2], 0.0, 1.0)

        lane = jax.lax.broadcasted_iota(
            jnp.int32, (1, F), 1).astype(jnp.float32)
        left = lane * (1.0 / F)
        right = left + (1.0 / F)
        ov = jnp.clip(jnp.minimum(e, right) - jnp.maximum(s, left), 0.0, None)
        not_skip = jnp.sum(ov) > 0.0
        n_col = jax.lax.broadcasted_iota(jnp.int32, (N, F), 0)
        s_b = jnp.broadcast_to(s, (N, F))
        e_b = jnp.broadcast_to(e, (N, F))

        for p in range(Pp):
            m = jnp.max(ov, axis=0, keepdims=True)
            idx = jnp.min(jnp.where(ov == m, n_col, N), axis=0, keepdims=True)
            oh = n_col == idx
            ts = jnp.sum(jnp.where(oh, s_b, 0.0), axis=0, keepdims=True)
            te = jnp.sum(jnp.where(oh, e_b, 0.0), axis=0, keepdims=True)
            pos = (m > 0.0) & not_skip
            ov = jnp.where(oh, -1.0, ov)

            z = pos.astype(jnp.float32)
            rw = 5.0 * z
            aw = 0.5 + 0.5 * z

            ps = raw_ref[bb, p, 0]
            pe = raw_ref[bb, p, 1]
            pc = raw_ref[bb, p, 2]
            d1 = jnp.abs(ps - ts)
            m1 = jnp.minimum(d1, 1.0)
            d2 = jnp.abs(pe - te)
            m2 = jnp.minimum(d2, 1.0)
            sl = m1 * (d1 - 0.5 * m1) + m2 * (d2 - 0.5 * m2)
            sp = jnp.maximum(pc, 0.0) + jnp.log1p(jnp.exp(-jnp.abs(pc)))
            contrib = rw * sl + aw * sp - z * pc
            main_b += jnp.sum(contrib)
            npos_b += jnp.float32(T) * jnp.sum(z)

    blk = jnp.concatenate(
        [jnp.full((1, 128), main_b, jnp.float32),
         jnp.full((1, 128), npos_b, jnp.float32),
         jnp.zeros((6, 128), jnp.float32)], axis=0)

    @pl.when(b == 0)
    def _():
        out_ref[...] = blk

    @pl.when(b != 0)
    def _():
        out_ref[...] = out_ref[...] + blk


@functools.partial(jax.jit, static_argnames=())
def kernel(raw_preds, gt_boxes):
    B, Pp, C, T, F = raw_preds.shape
    N = gt_boxes.shape[1]
    BB = 4
    out = pl.pallas_call(
        _loss_block,
        grid=(B // BB,),
        in_specs=[
            pl.BlockSpec((BB, Pp, C, T, F), lambda b: (b, 0, 0, 0, 0)),
            pl.BlockSpec((BB, N, 2), lambda b: (b, 0, 0)),
        ],
        out_specs=pl.BlockSpec((8, 128), lambda b: (0, 0)),
        out_shape=jax.ShapeDtypeStruct((8, 128), jnp.float32),
    )(raw_preds, gt_boxes)
    main = out[0, 0]
    n_pos = out[1, 0]
    return main / jnp.maximum(n_pos, 1.0)
